# baseline (device time: 84044 ns/iter reference)
import jax
import jax.numpy as jnp
from jax import lax
from jax.experimental import pallas as pl
from jax.experimental.pallas import tpu as pltpu

N_DEV = 32
B = 2
SQ = 256
SKV = 256
H_PER = 4
DH = 64
D_MODEL = 512
CHUNK = (B * SQ) // N_DEV
CHUNKS_PER_B = SQ // CHUNK


def kernel(x, Wq, K_ext, V_ext, Wo):
    def body(x_ref, wq_ref, k_hbm, v_hbm, wo_ref, out_ref,
             kv_ref, partial_ref, comm_ref, red_ref,
             kv_sems, send1, recv1, send2, recv2):
        my = lax.axis_index("i")

        kcopy = pltpu.make_async_copy(
            k_hbm.at[:, :, pl.ds(my * H_PER, H_PER), :], kv_ref.at[0],
            kv_sems.at[0])
        vcopy = pltpu.make_async_copy(
            v_hbm.at[:, :, pl.ds(my * H_PER, H_PER), :], kv_ref.at[1],
            kv_sems.at[1])
        kcopy.start()
        vcopy.start()

        barrier = pltpu.get_barrier_semaphore()
        for p in range(N_DEV):
            @pl.when(p != my)
            def _(p=p):
                pl.semaphore_signal(
                    barrier, inc=1,
                    device_id=(p,), device_id_type=pl.DeviceIdType.MESH,
                )
        pl.semaphore_wait(barrier, N_DEV - 1)

        qi = lax.broadcasted_iota(jnp.int32, (SQ, SKV), 0)
        ki = lax.broadcasted_iota(jnp.int32, (SQ, SKV), 1)
        mask = jnp.abs(qi - ki) <= 128

        wq = wq_ref[:, :]
        wo = wo_ref[:, :]
        qb_all = [
            jnp.dot(x_ref[b], wq, preferred_element_type=jnp.float32)
            for b in range(B)
        ]
        kcopy.wait()
        vcopy.wait()

        sends1 = []
        for b in range(B):
            ctxs = []
            for h in range(H_PER):
                qh = qb_all[b][:, h * DH:(h + 1) * DH]
                kh = kv_ref[0, b, :, h, :]
                vh = kv_ref[1, b, :, h, :]
                s = lax.dot_general(
                    qh, kh, (((1,), (1,)), ((), ())),
                    preferred_element_type=jnp.float32,
                ) * 0.125
                s = jnp.where(mask, s, -1e9)
                s = s - jnp.max(s, axis=1, keepdims=True)
                w = jnp.exp(s)
                w = w / jnp.sum(w, axis=1, keepdims=True)
                ctxs.append(jnp.dot(w, vh, preferred_element_type=jnp.float32))
            ctx_b = jnp.concatenate(ctxs, axis=1)
            pb = jnp.dot(ctx_b, wo, preferred_element_type=jnp.float32)
            for jj in range(CHUNKS_PER_B):
                partial_ref[b * CHUNKS_PER_B + jj, :, :] = \
                    pb[jj * CHUNK:(jj + 1) * CHUNK, :]

            for o in range(CHUNKS_PER_B):
                p = b * CHUNKS_PER_B + (my + o) % CHUNKS_PER_B
                rdma = pltpu.make_async_remote_copy(
                    src_ref=partial_ref.at[p],
                    dst_ref=comm_ref.at[my],
                    send_sem=send1.at[p],
                    recv_sem=recv1.at[my],
                    device_id=(p,), device_id_type=pl.DeviceIdType.MESH,
                )
                sends1.append((p, rdma))
                @pl.when(p != my)
                def _(rdma=rdma):
                    rdma.start()

        comm_ref[pl.ds(my, 1)] = partial_ref[pl.ds(my, 1)]

        for i in range(N_DEV):
            rdma = pltpu.make_async_remote_copy(
                src_ref=partial_ref.at[i],
                dst_ref=comm_ref.at[i],
                send_sem=send1.at[i],
                recv_sem=recv1.at[i],
                device_id=(i,), device_id_type=pl.DeviceIdType.MESH,
            )
            @pl.when(i != my)
            def _(rdma=rdma):
                rdma.wait_recv()

        red = jnp.sum(comm_ref[:, :, :], axis=0)
        red_ref[0, :, :] = red

        my_b = my // CHUNKS_PER_B
        my_r = (my % CHUNKS_PER_B) * CHUNK
        out_ref[pl.ds(my_b, 1), pl.ds(my_r, CHUNK), :] = red[None, :, :]

        sends2 = []
        for o in range(1, N_DEV):
            p = (my + o) % N_DEV
            rdma = pltpu.make_async_remote_copy(
                src_ref=red_ref,
                dst_ref=out_ref.at[pl.ds(my_b, 1), pl.ds(my_r, CHUNK), :],
                send_sem=send2.at[p],
                recv_sem=recv2.at[my],
                device_id=(p,), device_id_type=pl.DeviceIdType.MESH,
            )
            sends2.append(rdma)
            rdma.start()

        for i in range(N_DEV):
            rdma = pltpu.make_async_remote_copy(
                src_ref=red_ref,
                dst_ref=out_ref.at[
                    pl.ds(i // CHUNKS_PER_B, 1),
                    pl.ds((i % CHUNKS_PER_B) * CHUNK, CHUNK), :],
                send_sem=send2.at[i],
                recv_sem=recv2.at[i],
                device_id=(i,), device_id_type=pl.DeviceIdType.MESH,
            )
            @pl.when(i != my)
            def _(rdma=rdma):
                rdma.wait_recv()

        for p, rdma in sends1:
            @pl.when(p != my)
            def _(rdma=rdma):
                rdma.wait_send()
        for rdma in sends2:
            rdma.wait_send()

    return pl.pallas_call(
        body,
        out_shape=jax.ShapeDtypeStruct((B, SQ, D_MODEL), jnp.float32),
        in_specs=[
            pl.BlockSpec(memory_space=pltpu.VMEM),
            pl.BlockSpec(memory_space=pltpu.VMEM),
            pl.BlockSpec(memory_space=pl.ANY),
            pl.BlockSpec(memory_space=pl.ANY),
            pl.BlockSpec(memory_space=pltpu.VMEM),
        ],
        out_specs=pl.BlockSpec(memory_space=pltpu.VMEM),
        scratch_shapes=[
            pltpu.VMEM((2, B, SKV, H_PER, DH), jnp.float32),
            pltpu.VMEM((N_DEV, CHUNK, D_MODEL), jnp.float32),
            pltpu.VMEM((N_DEV, CHUNK, D_MODEL), jnp.float32),
            pltpu.VMEM((1, CHUNK, D_MODEL), jnp.float32),
            pltpu.SemaphoreType.DMA((2,)),
            pltpu.SemaphoreType.DMA((N_DEV,)),
            pltpu.SemaphoreType.DMA((N_DEV,)),
            pltpu.SemaphoreType.DMA((N_DEV,)),
            pltpu.SemaphoreType.DMA((N_DEV,)),
        ],
        compiler_params=pltpu.CompilerParams(collective_id=0),
    )(x, Wq, K_ext, V_ext, Wo)


# device time: 82380 ns/iter; 1.0202x vs baseline; 1.0202x over previous
import jax
import jax.numpy as jnp
from jax import lax
from jax.experimental import pallas as pl
from jax.experimental.pallas import tpu as pltpu

N_DEV = 32
B = 2
SQ = 256
SKV = 256
H_PER = 4
H_TOT = 128
DH = 64
D_MODEL = 512
CHUNK = (B * SQ) // N_DEV
CHUNKS_PER_B = SQ // CHUNK


def kernel(x, Wq, K_ext, V_ext, Wo):
    Kt = jnp.transpose(K_ext, (0, 1, 3, 2))
    Vt = jnp.transpose(V_ext, (0, 1, 3, 2))

    def body(x_ref, wq_ref, k_hbm, v_hbm, wo_ref, out_ref,
             kt_ref, vt_ref, partial_ref, comm_ref, red_ref,
             kv_sems, send1, recv1, send2, recv2):
        my = lax.axis_index("i")

        kcopy = pltpu.make_async_copy(k_hbm, kt_ref, kv_sems.at[0])
        vcopy = pltpu.make_async_copy(v_hbm, vt_ref, kv_sems.at[1])
        kcopy.start()
        vcopy.start()

        barrier = pltpu.get_barrier_semaphore()
        for p in range(N_DEV):
            @pl.when(p != my)
            def _(p=p):
                pl.semaphore_signal(
                    barrier, inc=1,
                    device_id=(p,), device_id_type=pl.DeviceIdType.MESH,
                )
        pl.semaphore_wait(barrier, N_DEV - 1)

        qi = lax.broadcasted_iota(jnp.int32, (SQ, SKV), 0)
        ki = lax.broadcasted_iota(jnp.int32, (SQ, SKV), 1)
        mask = jnp.abs(qi - ki) <= 128

        wq = wq_ref[:, :]
        wo = wo_ref[:, :]
        qb_all = [
            jnp.dot(x_ref[b], wq, preferred_element_type=jnp.float32)
            for b in range(B)
        ]
        kcopy.wait()
        vcopy.wait()

        sends1 = []
        for b in range(B):
            shift = H_TOT - my * H_PER
            kb = pltpu.roll(kt_ref[b], shift, 2)[:, :, :H_PER]
            vb = pltpu.roll(vt_ref[b], shift, 2)[:, :, :H_PER]
            ctxs = []
            for h in range(H_PER):
                qh = qb_all[b][:, h * DH:(h + 1) * DH]
                kh = kb[:, :, h]
                vh = vb[:, :, h]
                s = lax.dot_general(
                    qh, kh, (((1,), (1,)), ((), ())),
                    preferred_element_type=jnp.float32,
                ) * 0.125
                s = jnp.where(mask, s, -1e9)
                s = s - jnp.max(s, axis=1, keepdims=True)
                w = jnp.exp(s)
                w = w / jnp.sum(w, axis=1, keepdims=True)
                ctxs.append(jnp.dot(w, vh, preferred_element_type=jnp.float32))
            ctx_b = jnp.concatenate(ctxs, axis=1)
            pb = jnp.dot(ctx_b, wo, preferred_element_type=jnp.float32)
            for jj in range(CHUNKS_PER_B):
                partial_ref[b * CHUNKS_PER_B + jj, :, :] = \
                    pb[jj * CHUNK:(jj + 1) * CHUNK, :]

            for o in range(CHUNKS_PER_B):
                p = b * CHUNKS_PER_B + (my + o) % CHUNKS_PER_B
                rdma = pltpu.make_async_remote_copy(
                    src_ref=partial_ref.at[p],
                    dst_ref=comm_ref.at[my],
                    send_sem=send1.at[p],
                    recv_sem=recv1.at[my],
                    device_id=(p,), device_id_type=pl.DeviceIdType.MESH,
                )
                sends1.append((p, rdma))
                @pl.when(p != my)
                def _(rdma=rdma):
                    rdma.start()

        comm_ref[pl.ds(my, 1)] = partial_ref[pl.ds(my, 1)]

        for i in range(N_DEV):
            rdma = pltpu.make_async_remote_copy(
                src_ref=partial_ref.at[i],
                dst_ref=comm_ref.at[i],
                send_sem=send1.at[i],
                recv_sem=recv1.at[i],
                device_id=(i,), device_id_type=pl.DeviceIdType.MESH,
            )
            @pl.when(i != my)
            def _(rdma=rdma):
                rdma.wait_recv()

        red = jnp.sum(comm_ref[:, :, :], axis=0)
        red_ref[0, :, :] = red

        my_b = my // CHUNKS_PER_B
        my_r = (my % CHUNKS_PER_B) * CHUNK
        out_ref[pl.ds(my_b, 1), pl.ds(my_r, CHUNK), :] = red[None, :, :]

        sends2 = []
        for o in range(1, N_DEV):
            p = (my + o) % N_DEV
            rdma = pltpu.make_async_remote_copy(
                src_ref=red_ref,
                dst_ref=out_ref.at[pl.ds(my_b, 1), pl.ds(my_r, CHUNK), :],
                send_sem=send2.at[p],
                recv_sem=recv2.at[my],
                device_id=(p,), device_id_type=pl.DeviceIdType.MESH,
            )
            sends2.append(rdma)
            rdma.start()

        for i in range(N_DEV):
            rdma = pltpu.make_async_remote_copy(
                src_ref=red_ref,
                dst_ref=out_ref.at[
                    pl.ds(i // CHUNKS_PER_B, 1),
                    pl.ds((i % CHUNKS_PER_B) * CHUNK, CHUNK), :],
                send_sem=send2.at[i],
                recv_sem=recv2.at[i],
                device_id=(i,), device_id_type=pl.DeviceIdType.MESH,
            )
            @pl.when(i != my)
            def _(rdma=rdma):
                rdma.wait_recv()

        for p, rdma in sends1:
            @pl.when(p != my)
            def _(rdma=rdma):
                rdma.wait_send()
        for rdma in sends2:
            rdma.wait_send()

    return pl.pallas_call(
        body,
        out_shape=jax.ShapeDtypeStruct((B, SQ, D_MODEL), jnp.float32),
        in_specs=[
            pl.BlockSpec(memory_space=pltpu.VMEM),
            pl.BlockSpec(memory_space=pltpu.VMEM),
            pl.BlockSpec(memory_space=pl.ANY),
            pl.BlockSpec(memory_space=pl.ANY),
            pl.BlockSpec(memory_space=pltpu.VMEM),
        ],
        out_specs=pl.BlockSpec(memory_space=pltpu.VMEM),
        scratch_shapes=[
            pltpu.VMEM((B, SKV, DH, H_TOT), jnp.float32),
            pltpu.VMEM((B, SKV, DH, H_TOT), jnp.float32),
            pltpu.VMEM((N_DEV, CHUNK, D_MODEL), jnp.float32),
            pltpu.VMEM((N_DEV, CHUNK, D_MODEL), jnp.float32),
            pltpu.VMEM((1, CHUNK, D_MODEL), jnp.float32),
            pltpu.SemaphoreType.DMA((2,)),
            pltpu.SemaphoreType.DMA((N_DEV,)),
            pltpu.SemaphoreType.DMA((N_DEV,)),
            pltpu.SemaphoreType.DMA((N_DEV,)),
            pltpu.SemaphoreType.DMA((N_DEV,)),
        ],
        compiler_params=pltpu.CompilerParams(
            collective_id=0, vmem_limit_bytes=64 * 1024 * 1024),
    )(x, Wq, Kt, Vt, Wo)


# device time: 79745 ns/iter; 1.0539x vs baseline; 1.0330x over previous
import jax
import jax.numpy as jnp
from jax import lax
from jax.experimental import pallas as pl
from jax.experimental.pallas import tpu as pltpu

N_DEV = 32
B = 2
SQ = 256
SKV = 256
H_PER = 4
H_TOT = 128
DH = 64
D_MODEL = 512
CHUNK = (B * SQ) // N_DEV
CHUNKS_PER_B = SQ // CHUNK


def kernel(x, Wq, K_ext, V_ext, Wo):
    Kt = jnp.transpose(K_ext, (0, 1, 3, 2))
    Vt = jnp.transpose(V_ext, (0, 1, 3, 2))

    def body(x_ref, wq_ref, k_hbm, v_hbm, wo_ref, out_ref,
             kt_ref, vt_ref, partial_ref, comm_ref, red_ref,
             kv_sems, send1, recv1, send2, recv2):
        my = lax.axis_index("i")

        kcopy = pltpu.make_async_copy(k_hbm, kt_ref, kv_sems.at[0])
        vcopy = pltpu.make_async_copy(v_hbm, vt_ref, kv_sems.at[1])
        kcopy.start()
        vcopy.start()

        barrier = pltpu.get_barrier_semaphore()
        for p in range(N_DEV):
            @pl.when(p != my)
            def _(p=p):
                pl.semaphore_signal(
                    barrier, inc=1,
                    device_id=(p,), device_id_type=pl.DeviceIdType.MESH,
                )
        pl.semaphore_wait(barrier, N_DEV - 1)

        qi = lax.broadcasted_iota(jnp.int32, (SQ, SKV), 0)
        ki = lax.broadcasted_iota(jnp.int32, (SQ, SKV), 1)
        mask = jnp.abs(qi - ki) <= 128

        wq = wq_ref[:, :]
        wo = wo_ref[:, :]
        qb_all = [
            jnp.dot(x_ref[b], wq, preferred_element_type=jnp.float32)
            for b in range(B)
        ]
        kcopy.wait()
        vcopy.wait()

        sends1 = []
        for b in range(B):
            kb = kt_ref[b, :, :, 0:H_PER]
            vb = vt_ref[b, :, :, 0:H_PER]
            ctxs = []
            for h in range(H_PER):
                qh = qb_all[b][:, h * DH:(h + 1) * DH]
                kh = kb[:, :, h]
                vh = vb[:, :, h]
                s = lax.dot_general(
                    qh, kh, (((1,), (1,)), ((), ())),
                    preferred_element_type=jnp.float32,
                ) * 0.125
                s = jnp.where(mask, s, -1e9)
                s = s - jnp.max(s, axis=1, keepdims=True)
                w = jnp.exp(s)
                w = w / jnp.sum(w, axis=1, keepdims=True)
                ctxs.append(jnp.dot(w, vh, preferred_element_type=jnp.float32))
            ctx_b = jnp.concatenate(ctxs, axis=1)
            pb = jnp.dot(ctx_b, wo, preferred_element_type=jnp.float32)
            for jj in range(CHUNKS_PER_B):
                partial_ref[b * CHUNKS_PER_B + jj, :, :] = \
                    pb[jj * CHUNK:(jj + 1) * CHUNK, :]

            for o in range(CHUNKS_PER_B):
                p = b * CHUNKS_PER_B + (my + o) % CHUNKS_PER_B
                rdma = pltpu.make_async_remote_copy(
                    src_ref=partial_ref.at[p],
                    dst_ref=comm_ref.at[my],
                    send_sem=send1.at[p],
                    recv_sem=recv1.at[my],
                    device_id=(p,), device_id_type=pl.DeviceIdType.MESH,
                )
                sends1.append((p, rdma))
                @pl.when(p != my)
                def _(rdma=rdma):
                    rdma.start()

        comm_ref[pl.ds(my, 1)] = partial_ref[pl.ds(my, 1)]

        for i in range(N_DEV):
            rdma = pltpu.make_async_remote_copy(
                src_ref=partial_ref.at[i],
                dst_ref=comm_ref.at[i],
                send_sem=send1.at[i],
                recv_sem=recv1.at[i],
                device_id=(i,), device_id_type=pl.DeviceIdType.MESH,
            )
            @pl.when(i != my)
            def _(rdma=rdma):
                rdma.wait_recv()

        red = jnp.sum(comm_ref[:, :, :], axis=0)
        red_ref[0, :, :] = red

        my_b = my // CHUNKS_PER_B
        my_r = (my % CHUNKS_PER_B) * CHUNK
        out_ref[pl.ds(my_b, 1), pl.ds(my_r, CHUNK), :] = red[None, :, :]

        sends2 = []
        for o in range(1, N_DEV):
            p = (my + o) % N_DEV
            rdma = pltpu.make_async_remote_copy(
                src_ref=red_ref,
                dst_ref=out_ref.at[pl.ds(my_b, 1), pl.ds(my_r, CHUNK), :],
                send_sem=send2.at[p],
                recv_sem=recv2.at[my],
                device_id=(p,), device_id_type=pl.DeviceIdType.MESH,
            )
            sends2.append(rdma)
            rdma.start()

        for i in range(N_DEV):
            rdma = pltpu.make_async_remote_copy(
                src_ref=red_ref,
                dst_ref=out_ref.at[
                    pl.ds(i // CHUNKS_PER_B, 1),
                    pl.ds((i % CHUNKS_PER_B) * CHUNK, CHUNK), :],
                send_sem=send2.at[i],
                recv_sem=recv2.at[i],
                device_id=(i,), device_id_type=pl.DeviceIdType.MESH,
            )
            @pl.when(i != my)
            def _(rdma=rdma):
                rdma.wait_recv()

        for p, rdma in sends1:
            @pl.when(p != my)
            def _(rdma=rdma):
                rdma.wait_send()
        for rdma in sends2:
            rdma.wait_send()

    return pl.pallas_call(
        body,
        out_shape=jax.ShapeDtypeStruct((B, SQ, D_MODEL), jnp.float32),
        in_specs=[
            pl.BlockSpec(memory_space=pltpu.VMEM),
            pl.BlockSpec(memory_space=pltpu.VMEM),
            pl.BlockSpec(memory_space=pl.ANY),
            pl.BlockSpec(memory_space=pl.ANY),
            pl.BlockSpec(memory_space=pltpu.VMEM),
        ],
        out_specs=pl.BlockSpec(memory_space=pltpu.VMEM),
        scratch_shapes=[
            pltpu.VMEM((B, SKV, DH, H_TOT), jnp.float32),
            pltpu.VMEM((B, SKV, DH, H_TOT), jnp.float32),
            pltpu.VMEM((N_DEV, CHUNK, D_MODEL), jnp.float32),
            pltpu.VMEM((N_DEV, CHUNK, D_MODEL), jnp.float32),
            pltpu.VMEM((1, CHUNK, D_MODEL), jnp.float32),
            pltpu.SemaphoreType.DMA((2,)),
            pltpu.SemaphoreType.DMA((N_DEV,)),
            pltpu.SemaphoreType.DMA((N_DEV,)),
            pltpu.SemaphoreType.DMA((N_DEV,)),
            pltpu.SemaphoreType.DMA((N_DEV,)),
        ],
        compiler_params=pltpu.CompilerParams(
            collective_id=0, vmem_limit_bytes=64 * 1024 * 1024),
    )(x, Wq, Kt, Vt, Wo)


# device time: 78769 ns/iter; 1.0670x vs baseline; 1.0124x over previous
import jax
import jax.numpy as jnp
from jax import lax
from jax.experimental import pallas as pl
from jax.experimental.pallas import tpu as pltpu

N_DEV = 32
B = 2
SQ = 256
SKV = 256
H_PER = 4
H_TOT = 128
DH = 64
D_MODEL = 512
CHUNK = (B * SQ) // N_DEV
CHUNKS_PER_B = SQ // CHUNK
NKV_CHUNKS = 8
KV_ROWS = SKV // NKV_CHUNKS


def kernel(x, Wq, K_ext, V_ext, Wo):
    Kt = jnp.transpose(K_ext, (0, 1, 3, 2))
    Vt = jnp.transpose(V_ext, (0, 1, 3, 2))

    def body(x_ref, wq_ref, k_hbm, v_hbm, wo_ref, out_ref,
             kt_ref, vt_ref, partial_ref, comm_ref, red_ref,
             kv_sems, send1, recv1, send2, recv2):
        my = lax.axis_index("i")

        with jax.named_scope("kv_dma_start"):
            kv_copies = [[], []]
            for t, (src, dst) in enumerate(((k_hbm, kt_ref), (v_hbm, vt_ref))):
                for b in range(B):
                    for c in range(NKV_CHUNKS):
                        cp = pltpu.make_async_copy(
                            src.at[b, pl.ds(c * KV_ROWS, KV_ROWS)],
                            dst.at[b, pl.ds(c * KV_ROWS, KV_ROWS)],
                            kv_sems.at[(t * B + b) * NKV_CHUNKS + c])
                        cp.start()
                        kv_copies[b].append(cp)

        with jax.named_scope("barrier"):
            barrier = pltpu.get_barrier_semaphore()
            for p in range(N_DEV):
                @pl.when(p != my)
                def _(p=p):
                    pl.semaphore_signal(
                        barrier, inc=1,
                        device_id=(p,), device_id_type=pl.DeviceIdType.MESH,
                    )
            pl.semaphore_wait(barrier, N_DEV - 1)

        qi = lax.broadcasted_iota(jnp.int32, (SQ, SKV), 0)
        ki = lax.broadcasted_iota(jnp.int32, (SQ, SKV), 1)
        mask = jnp.abs(qi - ki) <= 128

        wq = wq_ref[:, :]
        wo = wo_ref[:, :]
        with jax.named_scope("qproj"):
            qb_all = [
                jnp.dot(x_ref[b], wq, preferred_element_type=jnp.float32)
                for b in range(B)
            ]
        sends1 = []
        for b in range(B):
            with jax.named_scope(f"kv_wait{b}"):
                for cp in kv_copies[b]:
                    cp.wait()
            with jax.named_scope(f"attn{b}"):
                shift = H_TOT - my * H_PER
                kb = pltpu.roll(kt_ref[b], shift, 2)[:, :, :H_PER]
                vb = pltpu.roll(vt_ref[b], shift, 2)[:, :, :H_PER]
                ctxs = []
                for h in range(H_PER):
                    qh = qb_all[b][:, h * DH:(h + 1) * DH]
                    kh = kb[:, :, h]
                    vh = vb[:, :, h]
                    s = lax.dot_general(
                        qh, kh, (((1,), (1,)), ((), ())),
                        preferred_element_type=jnp.float32,
                    ) * 0.125
                    s = jnp.where(mask, s, -1e9)
                    s = s - jnp.max(s, axis=1, keepdims=True)
                    w = jnp.exp(s)
                    w = w / jnp.sum(w, axis=1, keepdims=True)
                    ctxs.append(
                        jnp.dot(w, vh, preferred_element_type=jnp.float32))
                ctx_b = jnp.concatenate(ctxs, axis=1)
                pb = jnp.dot(ctx_b, wo, preferred_element_type=jnp.float32)
                for jj in range(CHUNKS_PER_B):
                    partial_ref[b * CHUNKS_PER_B + jj, :, :] = \
                        pb[jj * CHUNK:(jj + 1) * CHUNK, :]

            with jax.named_scope(f"send1b{b}"):
                for o in range(CHUNKS_PER_B):
                    p = b * CHUNKS_PER_B + (my + o) % CHUNKS_PER_B
                    rdma = pltpu.make_async_remote_copy(
                        src_ref=partial_ref.at[p],
                        dst_ref=comm_ref.at[my],
                        send_sem=send1.at[p],
                        recv_sem=recv1.at[my],
                        device_id=(p,), device_id_type=pl.DeviceIdType.MESH,
                    )
                    sends1.append((p, rdma))
                    @pl.when(p != my)
                    def _(rdma=rdma):
                        rdma.start()

        comm_ref[pl.ds(my, 1)] = partial_ref[pl.ds(my, 1)]

        with jax.named_scope("recv1"):
            for i in range(N_DEV):
                rdma = pltpu.make_async_remote_copy(
                    src_ref=partial_ref.at[i],
                    dst_ref=comm_ref.at[i],
                    send_sem=send1.at[i],
                    recv_sem=recv1.at[i],
                    device_id=(i,), device_id_type=pl.DeviceIdType.MESH,
                )
                @pl.when(i != my)
                def _(rdma=rdma):
                    rdma.wait_recv()

        with jax.named_scope("reduce"):
            red = jnp.sum(comm_ref[:, :, :], axis=0)
            red_ref[0, :, :] = red

            my_b = my // CHUNKS_PER_B
            my_r = (my % CHUNKS_PER_B) * CHUNK
            out_ref[pl.ds(my_b, 1), pl.ds(my_r, CHUNK), :] = red[None, :, :]

        with jax.named_scope("send2"):
            sends2 = []
            for o in range(1, N_DEV):
                p = (my + o) % N_DEV
                rdma = pltpu.make_async_remote_copy(
                    src_ref=red_ref,
                    dst_ref=out_ref.at[pl.ds(my_b, 1), pl.ds(my_r, CHUNK), :],
                    send_sem=send2.at[p],
                    recv_sem=recv2.at[my],
                    device_id=(p,), device_id_type=pl.DeviceIdType.MESH,
                )
                sends2.append(rdma)
                rdma.start()

        with jax.named_scope("recv2"):
            for i in range(N_DEV):
                rdma = pltpu.make_async_remote_copy(
                    src_ref=red_ref,
                    dst_ref=out_ref.at[
                        pl.ds(i // CHUNKS_PER_B, 1),
                        pl.ds((i % CHUNKS_PER_B) * CHUNK, CHUNK), :],
                    send_sem=send2.at[i],
                    recv_sem=recv2.at[i],
                    device_id=(i,), device_id_type=pl.DeviceIdType.MESH,
                )
                @pl.when(i != my)
                def _(rdma=rdma):
                    rdma.wait_recv()

        with jax.named_scope("drain"):
            for p, rdma in sends1:
                @pl.when(p != my)
                def _(rdma=rdma):
                    rdma.wait_send()
            for rdma in sends2:
                rdma.wait_send()

    return pl.pallas_call(
        body,
        out_shape=jax.ShapeDtypeStruct((B, SQ, D_MODEL), jnp.float32),
        in_specs=[
            pl.BlockSpec(memory_space=pltpu.VMEM),
            pl.BlockSpec(memory_space=pltpu.VMEM),
            pl.BlockSpec(memory_space=pl.ANY),
            pl.BlockSpec(memory_space=pl.ANY),
            pl.BlockSpec(memory_space=pltpu.VMEM),
        ],
        out_specs=pl.BlockSpec(memory_space=pltpu.VMEM),
        scratch_shapes=[
            pltpu.VMEM((B, SKV, DH, H_TOT), jnp.float32),
            pltpu.VMEM((B, SKV, DH, H_TOT), jnp.float32),
            pltpu.VMEM((N_DEV, CHUNK, D_MODEL), jnp.float32),
            pltpu.VMEM((N_DEV, CHUNK, D_MODEL), jnp.float32),
            pltpu.VMEM((1, CHUNK, D_MODEL), jnp.float32),
            pltpu.SemaphoreType.DMA((2 * B * NKV_CHUNKS,)),
            pltpu.SemaphoreType.DMA((N_DEV,)),
            pltpu.SemaphoreType.DMA((N_DEV,)),
            pltpu.SemaphoreType.DMA((N_DEV,)),
            pltpu.SemaphoreType.DMA((N_DEV,)),
        ],
        compiler_params=pltpu.CompilerParams(
            collective_id=0, vmem_limit_bytes=64 * 1024 * 1024),
    )(x, Wq, Kt, Vt, Wo)


# device time: 78575 ns/iter; 1.0696x vs baseline; 1.0025x over previous
import jax
import jax.numpy as jnp
from jax import lax
from jax.experimental import pallas as pl
from jax.experimental.pallas import tpu as pltpu

N_DEV = 32
B = 2
SQ = 256
SKV = 256
H_PER = 4
H_TOT = 128
DH = 64
D_MODEL = 512
CHUNK = (B * SQ) // N_DEV
CHUNKS_PER_B = SQ // CHUNK


def kernel(x, Wq, K_ext, V_ext, Wo):
    pos = lax.axis_index("i")
    Kt = jnp.transpose(K_ext, (0, 1, 3, 2))
    Vt = jnp.transpose(V_ext, (0, 1, 3, 2))
    K_sl = jnp.transpose(
        lax.dynamic_slice_in_dim(Kt, pos * H_PER, H_PER, axis=3), (0, 1, 3, 2))
    V_sl = jnp.transpose(
        lax.dynamic_slice_in_dim(Vt, pos * H_PER, H_PER, axis=3), (0, 1, 3, 2))

    def body(x_ref, wq_ref, k_ref, v_ref, wo_ref, out_ref,
             partial_ref, comm_ref, red_ref,
             send1, recv1, send2, recv2):
        my = lax.axis_index("i")

        with jax.named_scope("barrier"):
            barrier = pltpu.get_barrier_semaphore()
            for p in range(N_DEV):
                @pl.when(p != my)
                def _(p=p):
                    pl.semaphore_signal(
                        barrier, inc=1,
                        device_id=(p,), device_id_type=pl.DeviceIdType.MESH,
                    )
            pl.semaphore_wait(barrier, N_DEV - 1)

        qi = lax.broadcasted_iota(jnp.int32, (SQ, SKV), 0)
        ki = lax.broadcasted_iota(jnp.int32, (SQ, SKV), 1)
        mask = jnp.abs(qi - ki) <= 128

        wq = wq_ref[:, :]
        wo = wo_ref[:, :]
        with jax.named_scope("qproj"):
            qb_all = [
                jnp.dot(x_ref[b], wq, preferred_element_type=jnp.float32)
                for b in range(B)
            ]

        sends1 = []
        for b in range(B):
            with jax.named_scope(f"attn{b}"):
                ctxs = []
                for h in range(H_PER):
                    qh = qb_all[b][:, h * DH:(h + 1) * DH]
                    kh = k_ref[b, :, h, :]
                    vh = v_ref[b, :, h, :]
                    s = lax.dot_general(
                        qh, kh, (((1,), (1,)), ((), ())),
                        preferred_element_type=jnp.float32,
                    ) * 0.125
                    s = jnp.where(mask, s, -1e9)
                    s = s - jnp.max(s, axis=1, keepdims=True)
                    w = jnp.exp(s)
                    w = w / jnp.sum(w, axis=1, keepdims=True)
                    ctxs.append(
                        jnp.dot(w, vh, preferred_element_type=jnp.float32))
                ctx_b = jnp.concatenate(ctxs, axis=1)
                pb = jnp.dot(ctx_b, wo, preferred_element_type=jnp.float32)
                for jj in range(CHUNKS_PER_B):
                    partial_ref[b * CHUNKS_PER_B + jj, :, :] = \
                        pb[jj * CHUNK:(jj + 1) * CHUNK, :]

            with jax.named_scope(f"send1b{b}"):
                for o in range(CHUNKS_PER_B):
                    p = b * CHUNKS_PER_B + (my + o) % CHUNKS_PER_B
                    rdma = pltpu.make_async_remote_copy(
                        src_ref=partial_ref.at[p],
                        dst_ref=comm_ref.at[my],
                        send_sem=send1.at[p],
                        recv_sem=recv1.at[my],
                        device_id=(p,), device_id_type=pl.DeviceIdType.MESH,
                    )
                    sends1.append((p, rdma))
                    @pl.when(p != my)
                    def _(rdma=rdma):
                        rdma.start()

        comm_ref[pl.ds(my, 1)] = partial_ref[pl.ds(my, 1)]

        with jax.named_scope("recv1"):
            for i in range(N_DEV):
                rdma = pltpu.make_async_remote_copy(
                    src_ref=partial_ref.at[i],
                    dst_ref=comm_ref.at[i],
                    send_sem=send1.at[i],
                    recv_sem=recv1.at[i],
                    device_id=(i,), device_id_type=pl.DeviceIdType.MESH,
                )
                @pl.when(i != my)
                def _(rdma=rdma):
                    rdma.wait_recv()

        with jax.named_scope("reduce"):
            red = jnp.sum(comm_ref[:, :, :], axis=0)
            red_ref[0, :, :] = red

            my_b = my // CHUNKS_PER_B
            my_r = (my % CHUNKS_PER_B) * CHUNK
            out_ref[pl.ds(my_b, 1), pl.ds(my_r, CHUNK), :] = red[None, :, :]

        with jax.named_scope("send2"):
            sends2 = []
            for o in range(1, N_DEV):
                p = (my + o) % N_DEV
                rdma = pltpu.make_async_remote_copy(
                    src_ref=red_ref,
                    dst_ref=out_ref.at[pl.ds(my_b, 1), pl.ds(my_r, CHUNK), :],
                    send_sem=send2.at[p],
                    recv_sem=recv2.at[my],
                    device_id=(p,), device_id_type=pl.DeviceIdType.MESH,
                )
                sends2.append(rdma)
                rdma.start()

        with jax.named_scope("recv2"):
            for i in range(N_DEV):
                rdma = pltpu.make_async_remote_copy(
                    src_ref=red_ref,
                    dst_ref=out_ref.at[
                        pl.ds(i // CHUNKS_PER_B, 1),
                        pl.ds((i % CHUNKS_PER_B) * CHUNK, CHUNK), :],
                    send_sem=send2.at[i],
                    recv_sem=recv2.at[i],
                    device_id=(i,), device_id_type=pl.DeviceIdType.MESH,
                )
                @pl.when(i != my)
                def _(rdma=rdma):
                    rdma.wait_recv()

        with jax.named_scope("drain"):
            for p, rdma in sends1:
                @pl.when(p != my)
                def _(rdma=rdma):
                    rdma.wait_send()
            for rdma in sends2:
                rdma.wait_send()

    return pl.pallas_call(
        body,
        out_shape=jax.ShapeDtypeStruct((B, SQ, D_MODEL), jnp.float32),
        in_specs=[pl.BlockSpec(memory_space=pltpu.VMEM)] * 5,
        out_specs=pl.BlockSpec(memory_space=pltpu.VMEM),
        scratch_shapes=[
            pltpu.VMEM((N_DEV, CHUNK, D_MODEL), jnp.float32),
            pltpu.VMEM((N_DEV, CHUNK, D_MODEL), jnp.float32),
            pltpu.VMEM((1, CHUNK, D_MODEL), jnp.float32),
            pltpu.SemaphoreType.DMA((N_DEV,)),
            pltpu.SemaphoreType.DMA((N_DEV,)),
            pltpu.SemaphoreType.DMA((N_DEV,)),
            pltpu.SemaphoreType.DMA((N_DEV,)),
        ],
        compiler_params=pltpu.CompilerParams(collective_id=0),
    )(x, Wq, K_sl, V_sl, Wo)


# device time: 76484 ns/iter; 1.0988x vs baseline; 1.0273x over previous
import jax
import jax.numpy as jnp
from jax import lax
from jax.experimental import pallas as pl
from jax.experimental.pallas import tpu as pltpu

N_DEV = 32
B = 2
SQ = 256
SKV = 256
H_PER = 4
H_TOT = 128
DH = 64
D_MODEL = 512
CHUNK = (B * SQ) // N_DEV
CHUNKS_PER_B = SQ // CHUNK
KV_ROWS = SKV // N_DEV


def kernel(x, Wq, K_ext, V_ext, Wo):
    Kt = jnp.transpose(K_ext, (0, 1, 3, 2))
    Vt = jnp.transpose(V_ext, (0, 1, 3, 2))

    def body(x_ref, wq_ref, k_hbm, v_hbm, wo_ref, out_ref,
             kvread_ref, kvsend_ref, kvrecv_ref,
             partial_ref, comm_ref, red_ref,
             read_sems, sendkv, recvkv, send1, recv1, send2, recv2):
        my = lax.axis_index("i")

        with jax.named_scope("kv_read_start"):
            reads = []
            for t, src in enumerate((k_hbm, v_hbm)):
                for b in range(B):
                    cp = pltpu.make_async_copy(
                        src.at[b, pl.ds(my * KV_ROWS, KV_ROWS)],
                        kvread_ref.at[t, b],
                        read_sems.at[t * B + b])
                    cp.start()
                    reads.append(cp)

        with jax.named_scope("barrier"):
            barrier = pltpu.get_barrier_semaphore()
            for p in range(N_DEV):
                @pl.when(p != my)
                def _(p=p):
                    pl.semaphore_signal(
                        barrier, inc=1,
                        device_id=(p,), device_id_type=pl.DeviceIdType.MESH,
                    )
            pl.semaphore_wait(barrier, N_DEV - 1)

        with jax.named_scope("kv_extract"):
            for cp in reads:
                cp.wait()
            for t in range(2):
                for b in range(B):
                    val = kvread_ref[t, b]
                    tr = jnp.transpose(val, (0, 2, 1))
                    for p in range(N_DEV):
                        kvsend_ref[p, t, b] = \
                            tr[:, p * H_PER:(p + 1) * H_PER, :]
            kvrecv_ref[pl.ds(my, 1)] = kvsend_ref[pl.ds(my, 1)]

        with jax.named_scope("kv_send"):
            kv_sends = []
            for o in range(1, N_DEV):
                p = (my + o) % N_DEV
                rdma = pltpu.make_async_remote_copy(
                    src_ref=kvsend_ref.at[p],
                    dst_ref=kvrecv_ref.at[my],
                    send_sem=sendkv.at[p],
                    recv_sem=recvkv.at[my],
                    device_id=(p,), device_id_type=pl.DeviceIdType.MESH,
                )
                kv_sends.append(rdma)
                rdma.start()

        qi = lax.broadcasted_iota(jnp.int32, (SQ, SKV), 0)
        ki = lax.broadcasted_iota(jnp.int32, (SQ, SKV), 1)
        mask = jnp.abs(qi - ki) <= 128

        wq = wq_ref[:, :]
        wo = wo_ref[:, :]
        with jax.named_scope("qproj"):
            qb_all = [
                jnp.dot(x_ref[b], wq, preferred_element_type=jnp.float32)
                for b in range(B)
            ]

        with jax.named_scope("kv_recv"):
            for i in range(N_DEV):
                rdma = pltpu.make_async_remote_copy(
                    src_ref=kvsend_ref.at[i],
                    dst_ref=kvrecv_ref.at[i],
                    send_sem=sendkv.at[i],
                    recv_sem=recvkv.at[i],
                    device_id=(i,), device_id_type=pl.DeviceIdType.MESH,
                )
                @pl.when(i != my)
                def _(rdma=rdma):
                    rdma.wait_recv()

        sends1 = []
        for b in range(B):
            with jax.named_scope(f"attn{b}"):
                ctxs = []
                for h in range(H_PER):
                    qh = qb_all[b][:, h * DH:(h + 1) * DH]
                    kh = kvrecv_ref[:, 0, b, :, h, :].reshape(SKV, DH)
                    vh = kvrecv_ref[:, 1, b, :, h, :].reshape(SKV, DH)
                    s = lax.dot_general(
                        qh, kh, (((1,), (1,)), ((), ())),
                        preferred_element_type=jnp.float32,
                    ) * 0.125
                    s = jnp.where(mask, s, -1e9)
                    s = s - jnp.max(s, axis=1, keepdims=True)
                    w = jnp.exp(s)
                    w = w / jnp.sum(w, axis=1, keepdims=True)
                    ctxs.append(
                        jnp.dot(w, vh, preferred_element_type=jnp.float32))
                ctx_b = jnp.concatenate(ctxs, axis=1)
                pb = jnp.dot(ctx_b, wo, preferred_element_type=jnp.float32)
                for jj in range(CHUNKS_PER_B):
                    partial_ref[b * CHUNKS_PER_B + jj, :, :] = \
                        pb[jj * CHUNK:(jj + 1) * CHUNK, :]

            with jax.named_scope(f"send1b{b}"):
                for o in range(CHUNKS_PER_B):
                    p = b * CHUNKS_PER_B + (my + o) % CHUNKS_PER_B
                    rdma = pltpu.make_async_remote_copy(
                        src_ref=partial_ref.at[p],
                        dst_ref=comm_ref.at[my],
                        send_sem=send1.at[p],
                        recv_sem=recv1.at[my],
                        device_id=(p,), device_id_type=pl.DeviceIdType.MESH,
                    )
                    sends1.append((p, rdma))
                    @pl.when(p != my)
                    def _(rdma=rdma):
                        rdma.start()

        comm_ref[pl.ds(my, 1)] = partial_ref[pl.ds(my, 1)]

        with jax.named_scope("recv1"):
            for i in range(N_DEV):
                rdma = pltpu.make_async_remote_copy(
                    src_ref=partial_ref.at[i],
                    dst_ref=comm_ref.at[i],
                    send_sem=send1.at[i],
                    recv_sem=recv1.at[i],
                    device_id=(i,), device_id_type=pl.DeviceIdType.MESH,
                )
                @pl.when(i != my)
                def _(rdma=rdma):
                    rdma.wait_recv()

        with jax.named_scope("reduce"):
            red = jnp.sum(comm_ref[:, :, :], axis=0)
            red_ref[0, :, :] = red

            my_b = my // CHUNKS_PER_B
            my_r = (my % CHUNKS_PER_B) * CHUNK
            out_ref[pl.ds(my_b, 1), pl.ds(my_r, CHUNK), :] = red[None, :, :]

        with jax.named_scope("send2"):
            sends2 = []
            for o in range(1, N_DEV):
                p = (my + o) % N_DEV
                rdma = pltpu.make_async_remote_copy(
                    src_ref=red_ref,
                    dst_ref=out_ref.at[pl.ds(my_b, 1), pl.ds(my_r, CHUNK), :],
                    send_sem=send2.at[p],
                    recv_sem=recv2.at[my],
                    device_id=(p,), device_id_type=pl.DeviceIdType.MESH,
                )
                sends2.append(rdma)
                rdma.start()

        with jax.named_scope("recv2"):
            for i in range(N_DEV):
                rdma = pltpu.make_async_remote_copy(
                    src_ref=red_ref,
                    dst_ref=out_ref.at[
                        pl.ds(i // CHUNKS_PER_B, 1),
                        pl.ds((i % CHUNKS_PER_B) * CHUNK, CHUNK), :],
                    send_sem=send2.at[i],
                    recv_sem=recv2.at[i],
                    device_id=(i,), device_id_type=pl.DeviceIdType.MESH,
                )
                @pl.when(i != my)
                def _(rdma=rdma):
                    rdma.wait_recv()

        with jax.named_scope("drain"):
            for rdma in kv_sends:
                rdma.wait_send()
            for p, rdma in sends1:
                @pl.when(p != my)
                def _(rdma=rdma):
                    rdma.wait_send()
            for rdma in sends2:
                rdma.wait_send()

    return pl.pallas_call(
        body,
        out_shape=jax.ShapeDtypeStruct((B, SQ, D_MODEL), jnp.float32),
        in_specs=[
            pl.BlockSpec(memory_space=pltpu.VMEM),
            pl.BlockSpec(memory_space=pltpu.VMEM),
            pl.BlockSpec(memory_space=pl.ANY),
            pl.BlockSpec(memory_space=pl.ANY),
            pl.BlockSpec(memory_space=pltpu.VMEM),
        ],
        out_specs=pl.BlockSpec(memory_space=pltpu.VMEM),
        scratch_shapes=[
            pltpu.VMEM((2, B, KV_ROWS, DH, H_TOT), jnp.float32),
            pltpu.VMEM((N_DEV, 2, B, KV_ROWS, H_PER, DH), jnp.float32),
            pltpu.VMEM((N_DEV, 2, B, KV_ROWS, H_PER, DH), jnp.float32),
            pltpu.VMEM((N_DEV, CHUNK, D_MODEL), jnp.float32),
            pltpu.VMEM((N_DEV, CHUNK, D_MODEL), jnp.float32),
            pltpu.VMEM((1, CHUNK, D_MODEL), jnp.float32),
            pltpu.SemaphoreType.DMA((2 * B,)),
            pltpu.SemaphoreType.DMA((N_DEV,)),
            pltpu.SemaphoreType.DMA((N_DEV,)),
            pltpu.SemaphoreType.DMA((N_DEV,)),
            pltpu.SemaphoreType.DMA((N_DEV,)),
            pltpu.SemaphoreType.DMA((N_DEV,)),
            pltpu.SemaphoreType.DMA((N_DEV,)),
        ],
        compiler_params=pltpu.CompilerParams(collective_id=0),
    )(x, Wq, Kt, Vt, Wo)


# device time: 65934 ns/iter; 1.2747x vs baseline; 1.1600x over previous
import jax
import jax.numpy as jnp
from jax import lax
from jax.experimental import pallas as pl
from jax.experimental.pallas import tpu as pltpu

N_DEV = 32
B = 2
SQ = 256
SKV = 256
H_PER = 4
H_TOT = 128
DH = 64
D_MODEL = 512
CHUNK = (B * SQ) // N_DEV
CHUNKS_PER_B = SQ // CHUNK
KV_ROWS = SKV // N_DEV


def kernel(x, Wq, K_ext, V_ext, Wo):
    Kt = jnp.transpose(K_ext, (0, 1, 3, 2))
    Vt = jnp.transpose(V_ext, (0, 1, 3, 2))

    def body(x_ref, wq_ref, k_hbm, v_hbm, wo_ref, out_ref,
             kvread_ref, kvsend_ref, kvrecv_ref,
             partial_ref, comm_ref, red_ref,
             read_sems, sendkv, recvkv, send1, recv1, send2, recv2):
        my = lax.axis_index("i")

        with jax.named_scope("kv_read_start"):
            reads = []
            for t, src in enumerate((k_hbm, v_hbm)):
                for b in range(B):
                    cp = pltpu.make_async_copy(
                        src.at[b, pl.ds(my * KV_ROWS, KV_ROWS)],
                        kvread_ref.at[t, b],
                        read_sems.at[t * B + b])
                    cp.start()
                    reads.append(cp)

        with jax.named_scope("barrier"):
            barrier = pltpu.get_barrier_semaphore()
            for p in range(N_DEV):
                @pl.when(p != my)
                def _(p=p):
                    pl.semaphore_signal(
                        barrier, inc=1,
                        device_id=(p,), device_id_type=pl.DeviceIdType.MESH,
                    )
            pl.semaphore_wait(barrier, N_DEV - 1)

        with jax.named_scope("kv_extract"):
            for cp in reads:
                cp.wait()
            for t in range(2):
                for b in range(B):
                    val = kvread_ref[t, b]
                    tr = jnp.transpose(val, (0, 2, 1))
                    r0 = (t * B + b) * KV_ROWS
                    for p in range(N_DEV):
                        kvsend_ref[p, r0:r0 + KV_ROWS, :] = \
                            tr[:, p * H_PER:(p + 1) * H_PER, :].reshape(
                                KV_ROWS, H_PER * DH)
            kvrecv_ref[pl.ds(my, 1)] = kvsend_ref[pl.ds(my, 1)]

        with jax.named_scope("kv_send"):
            kv_sends = []
            for o in range(1, N_DEV):
                p = (my + o) % N_DEV
                rdma = pltpu.make_async_remote_copy(
                    src_ref=kvsend_ref.at[p],
                    dst_ref=kvrecv_ref.at[my],
                    send_sem=sendkv.at[p],
                    recv_sem=recvkv.at[my],
                    device_id=(p,), device_id_type=pl.DeviceIdType.MESH,
                )
                kv_sends.append(rdma)
                rdma.start()

        qi = lax.broadcasted_iota(jnp.int32, (SQ, SKV), 0)
        ki = lax.broadcasted_iota(jnp.int32, (SQ, SKV), 1)
        mask = jnp.abs(qi - ki) <= 128

        wq = wq_ref[:, :]
        wo = wo_ref[:, :]
        with jax.named_scope("qproj"):
            qb_all = [
                jnp.dot(x_ref[b], wq, preferred_element_type=jnp.float32)
                for b in range(B)
            ]

        with jax.named_scope("kv_recv"):
            for i in range(N_DEV):
                rdma = pltpu.make_async_remote_copy(
                    src_ref=kvsend_ref.at[i],
                    dst_ref=kvrecv_ref.at[i],
                    send_sem=sendkv.at[i],
                    recv_sem=recvkv.at[i],
                    device_id=(i,), device_id_type=pl.DeviceIdType.MESH,
                )
                @pl.when(i != my)
                def _(rdma=rdma):
                    rdma.wait_recv()

        sends1 = []
        for b in range(B):
            with jax.named_scope(f"attn{b}"):
                ctxs = []
                for h in range(H_PER):
                    qh = qb_all[b][:, h * DH:(h + 1) * DH]
                    kh = kvrecv_ref[
                        :, b * KV_ROWS:(b + 1) * KV_ROWS,
                        h * DH:(h + 1) * DH].reshape(SKV, DH)
                    vh = kvrecv_ref[
                        :, (B + b) * KV_ROWS:(B + b + 1) * KV_ROWS,
                        h * DH:(h + 1) * DH].reshape(SKV, DH)
                    s = lax.dot_general(
                        qh, kh, (((1,), (1,)), ((), ())),
                        preferred_element_type=jnp.float32,
                    ) * 0.125
                    s = jnp.where(mask, s, -1e9)
                    s = s - jnp.max(s, axis=1, keepdims=True)
                    w = jnp.exp(s)
                    w = w / jnp.sum(w, axis=1, keepdims=True)
                    ctxs.append(
                        jnp.dot(w, vh, preferred_element_type=jnp.float32))
                ctx_b = jnp.concatenate(ctxs, axis=1)
                pb = jnp.dot(ctx_b, wo, preferred_element_type=jnp.float32)
                for jj in range(CHUNKS_PER_B):
                    partial_ref[b * CHUNKS_PER_B + jj, :, :] = \
                        pb[jj * CHUNK:(jj + 1) * CHUNK, :]

            with jax.named_scope(f"send1b{b}"):
                for o in range(CHUNKS_PER_B):
                    p = b * CHUNKS_PER_B + (my + o) % CHUNKS_PER_B
                    rdma = pltpu.make_async_remote_copy(
                        src_ref=partial_ref.at[p],
                        dst_ref=comm_ref.at[my],
                        send_sem=send1.at[p],
                        recv_sem=recv1.at[my],
                        device_id=(p,), device_id_type=pl.DeviceIdType.MESH,
                    )
                    sends1.append((p, rdma))
                    @pl.when(p != my)
                    def _(rdma=rdma):
                        rdma.start()

        comm_ref[pl.ds(my, 1)] = partial_ref[pl.ds(my, 1)]

        with jax.named_scope("recv1"):
            for i in range(N_DEV):
                rdma = pltpu.make_async_remote_copy(
                    src_ref=partial_ref.at[i],
                    dst_ref=comm_ref.at[i],
                    send_sem=send1.at[i],
                    recv_sem=recv1.at[i],
                    device_id=(i,), device_id_type=pl.DeviceIdType.MESH,
                )
                @pl.when(i != my)
                def _(rdma=rdma):
                    rdma.wait_recv()

        with jax.named_scope("reduce"):
            red = jnp.sum(comm_ref[:, :, :], axis=0)
            red_ref[0, :, :] = red

            my_b = my // CHUNKS_PER_B
            my_r = (my % CHUNKS_PER_B) * CHUNK
            out_ref[pl.ds(my_b, 1), pl.ds(my_r, CHUNK), :] = red[None, :, :]

        with jax.named_scope("send2"):
            sends2 = []
            for o in range(1, N_DEV):
                p = (my + o) % N_DEV
                rdma = pltpu.make_async_remote_copy(
                    src_ref=red_ref,
                    dst_ref=out_ref.at[pl.ds(my_b, 1), pl.ds(my_r, CHUNK), :],
                    send_sem=send2.at[p],
                    recv_sem=recv2.at[my],
                    device_id=(p,), device_id_type=pl.DeviceIdType.MESH,
                )
                sends2.append(rdma)
                rdma.start()

        with jax.named_scope("recv2"):
            for i in range(N_DEV):
                rdma = pltpu.make_async_remote_copy(
                    src_ref=red_ref,
                    dst_ref=out_ref.at[
                        pl.ds(i // CHUNKS_PER_B, 1),
                        pl.ds((i % CHUNKS_PER_B) * CHUNK, CHUNK), :],
                    send_sem=send2.at[i],
                    recv_sem=recv2.at[i],
                    device_id=(i,), device_id_type=pl.DeviceIdType.MESH,
                )
                @pl.when(i != my)
                def _(rdma=rdma):
                    rdma.wait_recv()

        with jax.named_scope("drain"):
            for rdma in kv_sends:
                rdma.wait_send()
            for p, rdma in sends1:
                @pl.when(p != my)
                def _(rdma=rdma):
                    rdma.wait_send()
            for rdma in sends2:
                rdma.wait_send()

    return pl.pallas_call(
        body,
        out_shape=jax.ShapeDtypeStruct((B, SQ, D_MODEL), jnp.float32),
        in_specs=[
            pl.BlockSpec(memory_space=pltpu.VMEM),
            pl.BlockSpec(memory_space=pltpu.VMEM),
            pl.BlockSpec(memory_space=pl.ANY),
            pl.BlockSpec(memory_space=pl.ANY),
            pl.BlockSpec(memory_space=pltpu.VMEM),
        ],
        out_specs=pl.BlockSpec(memory_space=pltpu.VMEM),
        scratch_shapes=[
            pltpu.VMEM((2, B, KV_ROWS, DH, H_TOT), jnp.float32),
            pltpu.VMEM((N_DEV, 2 * B * KV_ROWS, H_PER * DH), jnp.float32),
            pltpu.VMEM((N_DEV, 2 * B * KV_ROWS, H_PER * DH), jnp.float32),
            pltpu.VMEM((N_DEV, CHUNK, D_MODEL), jnp.float32),
            pltpu.VMEM((N_DEV, CHUNK, D_MODEL), jnp.float32),
            pltpu.VMEM((1, CHUNK, D_MODEL), jnp.float32),
            pltpu.SemaphoreType.DMA((2 * B,)),
            pltpu.SemaphoreType.DMA((N_DEV,)),
            pltpu.SemaphoreType.DMA((N_DEV,)),
            pltpu.SemaphoreType.DMA((N_DEV,)),
            pltpu.SemaphoreType.DMA((N_DEV,)),
            pltpu.SemaphoreType.DMA((N_DEV,)),
            pltpu.SemaphoreType.DMA((N_DEV,)),
        ],
        compiler_params=pltpu.CompilerParams(collective_id=0),
    )(x, Wq, Kt, Vt, Wo)


# device time: 60288 ns/iter; 1.3940x vs baseline; 1.0937x over previous
import jax
import jax.numpy as jnp
from jax import lax
from jax.experimental import pallas as pl
from jax.experimental.pallas import tpu as pltpu

N_DEV = 32
B = 2
SQ = 256
SKV = 256
H_PER = 4
H_TOT = 128
DH = 64
D_MODEL = 512
CHUNK = (B * SQ) // N_DEV
CHUNKS_PER_B = SQ // CHUNK
KV_ROWS = SKV // N_DEV


def kernel(x, Wq, K_ext, V_ext, Wo):
    Kt = jnp.transpose(K_ext, (0, 1, 3, 2))
    Vt = jnp.transpose(V_ext, (0, 1, 3, 2))

    def body(x_hbm, wq_hbm, k_hbm, v_hbm, wo_hbm, out_ref,
             xv_ref, wqv_ref, wov_ref,
             kvread_ref, kvsend_ref, kvrecv_ref,
             partial_ref, comm_ref, red_ref,
             in_sems, read_sems, sendkv, recvkv, send1, recv1, send2, recv2):
        my = lax.axis_index("i")

        with jax.named_scope("in_dma_start"):
            in_copies = []
            for i, (src, dst) in enumerate(
                    ((x_hbm, xv_ref), (wq_hbm, wqv_ref), (wo_hbm, wov_ref))):
                cp = pltpu.make_async_copy(src, dst, in_sems.at[i])
                cp.start()
                in_copies.append(cp)

        with jax.named_scope("kv_read_start"):
            reads = []
            for t, src in enumerate((k_hbm, v_hbm)):
                for b in range(B):
                    cp = pltpu.make_async_copy(
                        src.at[b, pl.ds(my * KV_ROWS, KV_ROWS)],
                        kvread_ref.at[t, b],
                        read_sems.at[t * B + b])
                    cp.start()
                    reads.append(cp)

        with jax.named_scope("barrier"):
            barrier = pltpu.get_barrier_semaphore()
            for p in range(N_DEV):
                @pl.when(p != my)
                def _(p=p):
                    pl.semaphore_signal(
                        barrier, inc=1,
                        device_id=(p,), device_id_type=pl.DeviceIdType.MESH,
                    )
            pl.semaphore_wait(barrier, N_DEV - 1)

        with jax.named_scope("kv_extract"):
            for cp in reads:
                cp.wait()
            for t in range(2):
                for b in range(B):
                    val = kvread_ref[t, b]
                    tr = jnp.transpose(val, (0, 2, 1))
                    r0 = (t * B + b) * KV_ROWS
                    for p in range(N_DEV):
                        kvsend_ref[p, r0:r0 + KV_ROWS, :] = \
                            tr[:, p * H_PER:(p + 1) * H_PER, :].reshape(
                                KV_ROWS, H_PER * DH)
            kvrecv_ref[pl.ds(my, 1)] = kvsend_ref[pl.ds(my, 1)]

        with jax.named_scope("kv_send"):
            kv_sends = []
            for o in range(1, N_DEV):
                p = (my + o) % N_DEV
                rdma = pltpu.make_async_remote_copy(
                    src_ref=kvsend_ref.at[p],
                    dst_ref=kvrecv_ref.at[my],
                    send_sem=sendkv.at[p],
                    recv_sem=recvkv.at[my],
                    device_id=(p,), device_id_type=pl.DeviceIdType.MESH,
                )
                kv_sends.append(rdma)
                rdma.start()

        qi = lax.broadcasted_iota(jnp.int32, (SQ, SKV), 0)
        ki = lax.broadcasted_iota(jnp.int32, (SQ, SKV), 1)
        mask = jnp.abs(qi - ki) <= 128

        with jax.named_scope("in_dma_wait"):
            for cp in in_copies:
                cp.wait()
        wq = wqv_ref[:, :]
        wo = wov_ref[:, :]
        with jax.named_scope("qproj"):
            qb_all = [
                jnp.dot(xv_ref[b], wq, preferred_element_type=jnp.float32)
                for b in range(B)
            ]

        with jax.named_scope("kv_recv"):
            for i in range(N_DEV):
                rdma = pltpu.make_async_remote_copy(
                    src_ref=kvsend_ref.at[i],
                    dst_ref=kvrecv_ref.at[i],
                    send_sem=sendkv.at[i],
                    recv_sem=recvkv.at[i],
                    device_id=(i,), device_id_type=pl.DeviceIdType.MESH,
                )
                @pl.when(i != my)
                def _(rdma=rdma):
                    rdma.wait_recv()

        sends1 = []
        for b in range(B):
            with jax.named_scope(f"attn{b}"):
                ctxs = []
                for h in range(H_PER):
                    qh = qb_all[b][:, h * DH:(h + 1) * DH]
                    kh = kvrecv_ref[
                        :, b * KV_ROWS:(b + 1) * KV_ROWS,
                        h * DH:(h + 1) * DH].reshape(SKV, DH)
                    vh = kvrecv_ref[
                        :, (B + b) * KV_ROWS:(B + b + 1) * KV_ROWS,
                        h * DH:(h + 1) * DH].reshape(SKV, DH)
                    s = lax.dot_general(
                        qh, kh, (((1,), (1,)), ((), ())),
                        preferred_element_type=jnp.float32,
                    ) * 0.125
                    s = jnp.where(mask, s, -1e9)
                    s = s - jnp.max(s, axis=1, keepdims=True)
                    w = jnp.exp(s)
                    w = w / jnp.sum(w, axis=1, keepdims=True)
                    ctxs.append(
                        jnp.dot(w, vh, preferred_element_type=jnp.float32))
                ctx_b = jnp.concatenate(ctxs, axis=1)
                pb = jnp.dot(ctx_b, wo, preferred_element_type=jnp.float32)
                for jj in range(CHUNKS_PER_B):
                    partial_ref[b * CHUNKS_PER_B + jj, :, :] = \
                        pb[jj * CHUNK:(jj + 1) * CHUNK, :]

            with jax.named_scope(f"send1b{b}"):
                for o in range(CHUNKS_PER_B):
                    p = b * CHUNKS_PER_B + (my + o) % CHUNKS_PER_B
                    rdma = pltpu.make_async_remote_copy(
                        src_ref=partial_ref.at[p],
                        dst_ref=comm_ref.at[my],
                        send_sem=send1.at[p],
                        recv_sem=recv1.at[my],
                        device_id=(p,), device_id_type=pl.DeviceIdType.MESH,
                    )
                    sends1.append((p, rdma))
                    @pl.when(p != my)
                    def _(rdma=rdma):
                        rdma.start()

        comm_ref[pl.ds(my, 1)] = partial_ref[pl.ds(my, 1)]

        with jax.named_scope("recv1"):
            for i in range(N_DEV):
                rdma = pltpu.make_async_remote_copy(
                    src_ref=partial_ref.at[i],
                    dst_ref=comm_ref.at[i],
                    send_sem=send1.at[i],
                    recv_sem=recv1.at[i],
                    device_id=(i,), device_id_type=pl.DeviceIdType.MESH,
                )
                @pl.when(i != my)
                def _(rdma=rdma):
                    rdma.wait_recv()

        with jax.named_scope("reduce"):
            red = jnp.sum(comm_ref[:, :, :], axis=0)
            red_ref[0, :, :] = red

            my_b = my // CHUNKS_PER_B
            my_r = (my % CHUNKS_PER_B) * CHUNK
            out_ref[pl.ds(my_b, 1), pl.ds(my_r, CHUNK), :] = red[None, :, :]

        with jax.named_scope("send2"):
            sends2 = []
            for o in range(1, N_DEV):
                p = (my + o) % N_DEV
                rdma = pltpu.make_async_remote_copy(
                    src_ref=red_ref,
                    dst_ref=out_ref.at[pl.ds(my_b, 1), pl.ds(my_r, CHUNK), :],
                    send_sem=send2.at[p],
                    recv_sem=recv2.at[my],
                    device_id=(p,), device_id_type=pl.DeviceIdType.MESH,
                )
                sends2.append(rdma)
                rdma.start()

        with jax.named_scope("recv2"):
            for i in range(N_DEV):
                rdma = pltpu.make_async_remote_copy(
                    src_ref=red_ref,
                    dst_ref=out_ref.at[
                        pl.ds(i // CHUNKS_PER_B, 1),
                        pl.ds((i % CHUNKS_PER_B) * CHUNK, CHUNK), :],
                    send_sem=send2.at[i],
                    recv_sem=recv2.at[i],
                    device_id=(i,), device_id_type=pl.DeviceIdType.MESH,
                )
                @pl.when(i != my)
                def _(rdma=rdma):
                    rdma.wait_recv()

        with jax.named_scope("drain"):
            for rdma in kv_sends:
                rdma.wait_send()
            for p, rdma in sends1:
                @pl.when(p != my)
                def _(rdma=rdma):
                    rdma.wait_send()
            for rdma in sends2:
                rdma.wait_send()

    return pl.pallas_call(
        body,
        out_shape=jax.ShapeDtypeStruct((B, SQ, D_MODEL), jnp.float32),
        in_specs=[pl.BlockSpec(memory_space=pl.ANY)] * 5,
        out_specs=pl.BlockSpec(memory_space=pltpu.VMEM),
        scratch_shapes=[
            pltpu.VMEM((B, SQ, D_MODEL), jnp.float32),
            pltpu.VMEM((D_MODEL, H_PER * DH), jnp.float32),
            pltpu.VMEM((H_PER * DH, D_MODEL), jnp.float32),
            pltpu.VMEM((2, B, KV_ROWS, DH, H_TOT), jnp.float32),
            pltpu.VMEM((N_DEV, 2 * B * KV_ROWS, H_PER * DH), jnp.float32),
            pltpu.VMEM((N_DEV, 2 * B * KV_ROWS, H_PER * DH), jnp.float32),
            pltpu.VMEM((N_DEV, CHUNK, D_MODEL), jnp.float32),
            pltpu.VMEM((N_DEV, CHUNK, D_MODEL), jnp.float32),
            pltpu.VMEM((1, CHUNK, D_MODEL), jnp.float32),
            pltpu.SemaphoreType.DMA((3,)),
            pltpu.SemaphoreType.DMA((2 * B,)),
            pltpu.SemaphoreType.DMA((N_DEV,)),
            pltpu.SemaphoreType.DMA((N_DEV,)),
            pltpu.SemaphoreType.DMA((N_DEV,)),
            pltpu.SemaphoreType.DMA((N_DEV,)),
            pltpu.SemaphoreType.DMA((N_DEV,)),
            pltpu.SemaphoreType.DMA((N_DEV,)),
        ],
        compiler_params=pltpu.CompilerParams(collective_id=0),
    )(x, Wq, Kt, Vt, Wo)
